# baseline (speedup 1.0000x reference)
import functools

import jax
import jax.numpy as jnp
from jax import lax
from jax.experimental import pallas as pl
from jax.experimental.pallas import tpu as pltpu
from jax.experimental.pallas import tpu_sc as plsc

_EMB_DIM = 128
_NUM_SEG = 26
_NUM_PAIR = _NUM_SEG // 2
_BATCH = 16384
_ROW = _NUM_SEG * _EMB_DIM
_CHUNK_ROWS = 64


def _permute_sc(pooled_embs):
    info = plsc.get_sparse_core_info()
    num_workers = info.num_cores * info.num_subcores
    rows_per_w = _BATCH // num_workers
    n_rchunks = rows_per_w // _CHUNK_ROWS
    mesh = plsc.VectorSubcoreMesh(core_axis_name="c", subcore_axis_name="s")

    @functools.partial(
        pl.kernel,
        mesh=mesh,
        out_type=jax.ShapeDtypeStruct((_BATCH, _ROW), jnp.float32),
        scratch_types=[
            pltpu.VMEM((_CHUNK_ROWS, 2 * _EMB_DIM), jnp.float32),
            pltpu.VMEM((_CHUNK_ROWS, 2 * _EMB_DIM), jnp.float32),
            pltpu.VMEM((_CHUNK_ROWS, 2 * _EMB_DIM), jnp.float32),
            pltpu.VMEM((_CHUNK_ROWS, 2 * _EMB_DIM), jnp.float32),
            pltpu.SemaphoreType.DMA,
            pltpu.SemaphoreType.DMA,
            pltpu.SemaphoreType.DMA,
            pltpu.SemaphoreType.DMA,
            pltpu.SemaphoreType.DMA,
            pltpu.SemaphoreType.DMA,
            pltpu.SemaphoreType.DMA,
            pltpu.SemaphoreType.DMA,
        ],
    )
    def k(in_hbm, out_hbm, b0, b1, b2, b3, g0, g1, g2, g3, s0, s1, s2, s3):
        wid = lax.axis_index("s") * info.num_cores + lax.axis_index("c")
        row_base = wid * rows_per_w

        nbuf = 4
        bufs = (b0, b1, b2, b3)
        gsems = (g0, g1, g2, g3)
        ssems = (s0, s1, s2, s3)
        steps = [
            (a, c) for a in range(_NUM_PAIR) for c in range(n_rchunks)
        ]
        n_steps = len(steps)

        def gather(t):
            a, c = steps[t]
            src_lo = (_NUM_SEG - 2 - 2 * a) * _EMB_DIM
            h = pltpu.make_async_copy(
                in_hbm.at[
                    pl.ds(row_base + c * _CHUNK_ROWS, _CHUNK_ROWS),
                    pl.ds(src_lo, 2 * _EMB_DIM),
                ],
                bufs[t % nbuf],
                gsems[t % nbuf],
            )
            h.start()
            return h

        def scatter(t):
            a, c = steps[t]
            rows = pl.ds(row_base + c * _CHUNK_ROWS, _CHUNK_ROWS)
            hs = []
            for half in range(2):
                dst = (2 * a + 1 - half) * _EMB_DIM
                h = pltpu.make_async_copy(
                    bufs[t % nbuf].at[:, pl.ds(half * _EMB_DIM, _EMB_DIM)],
                    out_hbm.at[rows, pl.ds(dst, _EMB_DIM)],
                    ssems[t % nbuf],
                )
                h.start()
                hs.append(h)
            return hs

        g_pend = {}
        s_pend = {}
        g_pend[0] = gather(0)
        g_pend[1] = gather(1)
        for t in range(n_steps):
            g_pend.pop(t).wait()
            s_pend[t] = scatter(t)
            u = t + 2
            if u < n_steps:
                if u - nbuf in s_pend:
                    for h in s_pend.pop(u - nbuf):
                        h.wait()
                g_pend[u] = gather(u)
        for t in sorted(s_pend):
            for h in s_pend.pop(t):
                h.wait()

    return k(pooled_embs)


def kernel(pooled_embs):
    return _permute_sc(pooled_embs)


# speedup vs baseline: 1.0462x; 1.0462x over previous
import functools

import jax
import jax.numpy as jnp
from jax import lax
from jax.experimental import pallas as pl
from jax.experimental.pallas import tpu as pltpu
from jax.experimental.pallas import tpu_sc as plsc

_EMB_DIM = 128
_NUM_SEG = 26
_PERM = tuple(range(_NUM_SEG - 1, -1, -1))
_BATCH = 16384
_ROW = _NUM_SEG * _EMB_DIM
_CHUNK_ROWS = 128
_NBUF = 6
_DEPTH = 3


def _permute_sc(pooled_embs):
    info = plsc.get_sparse_core_info()
    num_workers = info.num_cores * info.num_subcores
    rows_per_w = _BATCH // num_workers
    n_rchunks = rows_per_w // _CHUNK_ROWS
    mesh = plsc.VectorSubcoreMesh(core_axis_name="c", subcore_axis_name="s")

    @functools.partial(
        pl.kernel,
        mesh=mesh,
        out_type=jax.ShapeDtypeStruct((_BATCH, _ROW), jnp.float32),
        scratch_types=(
            [pltpu.VMEM((_CHUNK_ROWS, _EMB_DIM), jnp.float32)] * _NBUF
            + [pltpu.SemaphoreType.DMA] * (2 * _NBUF)
        ),
    )
    def k(in_hbm, out_hbm, *scr):
        bufs = scr[:_NBUF]
        gsems = scr[_NBUF : 2 * _NBUF]
        ssems = scr[2 * _NBUF :]
        wid = lax.axis_index("s") * info.num_cores + lax.axis_index("c")
        row_base = wid * rows_per_w

        steps = [
            (j, c) for j in range(_NUM_SEG) for c in range(n_rchunks)
        ]
        n_steps = len(steps)

        def gather(t):
            j, c = steps[t]
            src = _PERM[j]
            h = pltpu.make_async_copy(
                in_hbm.at[
                    pl.ds(row_base + c * _CHUNK_ROWS, _CHUNK_ROWS),
                    pl.ds(src * _EMB_DIM, _EMB_DIM),
                ],
                bufs[t % _NBUF],
                gsems[t % _NBUF],
            )
            h.start()
            return h

        def scatter(t):
            j, c = steps[t]
            h = pltpu.make_async_copy(
                bufs[t % _NBUF],
                out_hbm.at[
                    pl.ds(row_base + c * _CHUNK_ROWS, _CHUNK_ROWS),
                    pl.ds(j * _EMB_DIM, _EMB_DIM),
                ],
                ssems[t % _NBUF],
            )
            h.start()
            return h

        g_pend = {}
        s_pend = {}
        for t in range(_DEPTH):
            g_pend[t] = gather(t)
        for t in range(n_steps):
            g_pend.pop(t).wait()
            s_pend[t] = scatter(t)
            u = t + _DEPTH
            if u < n_steps:
                if u - _NBUF in s_pend:
                    s_pend.pop(u - _NBUF).wait()
                g_pend[u] = gather(u)
        for t in sorted(s_pend):
            s_pend.pop(t).wait()

    return k(pooled_embs)


def kernel(pooled_embs):
    return _permute_sc(pooled_embs)


# speedup vs baseline: 1.0592x; 1.0124x over previous
import functools

import jax
import jax.numpy as jnp
from jax import lax
from jax.experimental import pallas as pl
from jax.experimental.pallas import tpu as pltpu
from jax.experimental.pallas import tpu_sc as plsc

_EMB_DIM = 128
_NUM_SEG = 26
_BATCH = 16384
_ROW = _NUM_SEG * _EMB_DIM
_CHUNK_ROWS = 128
_NBUF = 4


def _permute_sc(pooled_embs):
    info = plsc.get_sparse_core_info()
    num_workers = info.num_cores * info.num_subcores
    rows_per_w = _BATCH // num_workers
    n_rchunks = rows_per_w // _CHUNK_ROWS
    assert n_rchunks == _NBUF
    mesh = plsc.VectorSubcoreMesh(core_axis_name="c", subcore_axis_name="s")

    @functools.partial(
        pl.kernel,
        mesh=mesh,
        out_type=jax.ShapeDtypeStruct((_BATCH, _ROW), jnp.float32),
        scratch_types=(
            [pltpu.VMEM((_CHUNK_ROWS, _EMB_DIM), jnp.float32)] * _NBUF
            + [pltpu.SemaphoreType.DMA] * (2 * _NBUF)
        ),
    )
    def k(in_hbm, out_hbm, *scr):
        bufs = scr[:_NBUF]
        gsems = scr[_NBUF : 2 * _NBUF]
        ssems = scr[2 * _NBUF :]
        wid = lax.axis_index("s") * info.num_cores + lax.axis_index("c")
        row_base = wid * rows_per_w

        def gather(j, c, slot):
            src_col = (_NUM_SEG - 1 - j) * _EMB_DIM
            h = pltpu.make_async_copy(
                in_hbm.at[
                    pl.ds(row_base + c * _CHUNK_ROWS, _CHUNK_ROWS),
                    pl.ds(src_col, _EMB_DIM),
                ],
                bufs[slot],
                gsems[slot],
            )
            h.start()
            return h

        def scatter(j, c, slot):
            h = pltpu.make_async_copy(
                bufs[slot],
                out_hbm.at[
                    pl.ds(row_base + c * _CHUNK_ROWS, _CHUNK_ROWS),
                    pl.ds(j * _EMB_DIM, _EMB_DIM),
                ],
                ssems[slot],
            )
            h.start()
            return h

        dummy_in = in_hbm.at[pl.ds(0, _CHUNK_ROWS), pl.ds(0, _EMB_DIM)]
        dummy_out = out_hbm.at[pl.ds(0, _CHUNK_ROWS), pl.ds(0, _EMB_DIM)]

        def wait_gather(slot):
            pltpu.make_async_copy(dummy_in, bufs[slot], gsems[slot]).wait()

        def wait_scatter(slot):
            pltpu.make_async_copy(bufs[slot], dummy_out, ssems[slot]).wait()

        gather(0, 0, 0)
        gather(0, 1, 1)
        wait_gather(0)
        scatter(0, 0, 0)
        gather(0, 2, 2)
        wait_gather(1)
        scatter(0, 1, 1)
        gather(0, 3, 3)

        def body(kk, carry):
            for b in range(4):
                if b < 2:
                    j, c = kk, 2 + b
                else:
                    j, c = kk + 1, b - 2
                slot = (2 + b) % 4
                wait_gather(slot)
                scatter(j, c, slot)
                wait_scatter(b)
                gather(kk + 1, b, b)
            return carry

        lax.fori_loop(0, _NUM_SEG - 1, body, 0)

        wait_gather(2)
        scatter(_NUM_SEG - 1, 2, 2)
        wait_gather(3)
        scatter(_NUM_SEG - 1, 3, 3)
        for slot in range(4):
            wait_scatter(slot)

    return k(pooled_embs)


def kernel(pooled_embs):
    return _permute_sc(pooled_embs)
